# Initial kernel scaffold; baseline (speedup 1.0000x reference)
import jax
import jax.numpy as jnp
from jax.experimental import pallas as pl


def kernel(gt, cls_gt, logits_1, logits_2, selector, it):
    raise NotImplementedError("write your pallas kernel here")


# speedup vs baseline: 93.6909x; 93.6909x over previous
import jax
import jax.numpy as jnp
from jax import lax
from jax.experimental import pallas as pl
from jax.experimental.pallas import tpu as pltpu

_H = 384
_W = 384
_N = _H * _W
_K = int(_N * 0.15)
_START_WARM = 20000


def _raw_ce(L, tgt, full):
    l0, l1, l2 = L[0], L[1], L[2]
    if full:
        m = jnp.maximum(jnp.maximum(l0, l1), l2)
        lse = jnp.log(jnp.exp(l0 - m) + jnp.exp(l1 - m) + jnp.exp(l2 - m)) + m
        lt = jnp.where(tgt == 0, l0, jnp.where(tgt == 1, l1, l2))
    else:
        m = jnp.maximum(l0, l1)
        lse = jnp.log(jnp.exp(l0 - m) + jnp.exp(l1 - m)) + m
        lt = jnp.where(tgt >= 1, l1, l0)
    return jnp.maximum(lse - lt, 0.0)


def _ce_topk_kernel(sel_ref, cls_ref, l1_ref, l2_ref, topk_ref, tot_ref,
                    raw_ref):
    j = pl.program_id(0)
    sel_full = sel_ref[j, 1] > 0.5

    for i in (1, 2):
        L = l1_ref[0] if i == 1 else l2_ref[0]
        tgt = cls_ref[0, i]
        raw = lax.cond(sel_full,
                       lambda L=L, tgt=tgt: _raw_ce(L, tgt, True),
                       lambda L=L, tgt=tgt: _raw_ce(L, tgt, False))
        raw_ref[(i - 1) * 4 + j] = raw

    @pl.when(j == 3)
    def _search():
        raw = raw_ref[...]
        kf = jnp.float32(_K)

        def body(t, T):
            cand = T | (jnp.int32(1) << (jnp.int32(30) - t))
            v = lax.bitcast_convert_type(cand, jnp.float32)
            cnt = jnp.sum((raw >= v).astype(jnp.float32), axis=(1, 2),
                          keepdims=True)
            return jnp.where(cnt >= kf, cand, T)

        T = lax.fori_loop(0, 31, body, jnp.zeros((8, 1, 1), jnp.int32))
        v = lax.bitcast_convert_type(T, jnp.float32)
        gt = (raw > v).astype(jnp.float32)
        s = jnp.sum(raw * gt, axis=(1, 2), keepdims=True)
        c = jnp.sum(gt, axis=(1, 2), keepdims=True)
        tk = (s + (kf - c) * v).reshape(8, 1)
        topk_ref[...] = jnp.broadcast_to(tk, (8, 128))
        tot = jnp.sum(raw, axis=(1, 2)).reshape(8, 1)
        tot_ref[...] = jnp.broadcast_to(tot, (8, 128))


def _run(cls_gt, logits_1, logits_2, selector, interpret=False):
    return pl.pallas_call(
        _ce_topk_kernel,
        grid=(4,),
        in_specs=[
            pl.BlockSpec(memory_space=pltpu.SMEM),
            pl.BlockSpec((1, 3, _H, _W), lambda j: (j, 0, 0, 0)),
            pl.BlockSpec((1, 3, _H, _W), lambda j: (j, 0, 0, 0)),
            pl.BlockSpec((1, 3, _H, _W), lambda j: (j, 0, 0, 0)),
        ],
        out_specs=[
            pl.BlockSpec((8, 128), lambda j: (0, 0)),
            pl.BlockSpec((8, 128), lambda j: (0, 0)),
        ],
        out_shape=[
            jax.ShapeDtypeStruct((8, 128), jnp.float32),
            jax.ShapeDtypeStruct((8, 128), jnp.float32),
        ],
        scratch_shapes=[pltpu.VMEM((8, _H, _W), jnp.float32)],
        interpret=interpret,
    )(selector, cls_gt, logits_1, logits_2)


def kernel(gt, cls_gt, logits_1, logits_2, selector, it):
    topk, tot = _run(cls_gt, logits_1, logits_2, selector)
    per_topk = topk[:, 0] / jnp.float32(_K)
    per_tot = tot[:, 0] / jnp.float32(_N)
    per = jnp.where(it < _START_WARM, per_tot, per_topk)
    return jnp.sum(per) * jnp.float32(0.25)
